# baseline (speedup 1.0000x reference)
import functools

import jax
import jax.numpy as jnp
from jax import lax
from jax.experimental import pallas as pl
from jax.experimental.pallas import tpu as pltpu
from jax.experimental.pallas import tpu_sc as plsc

S, L, V, A = 4, 3, 8, 10
B = 16384
TABLE = S * L * V * A

_info = plsc.get_sparse_core_info()
_NS, _LANES = _info.num_subcores, _info.num_lanes
_NW = _NS
_BPW = B // _NW
_STEPS = _BPW // _LANES


def _body(table_hbm, src_hbm, lab_hbm, vt_hbm, ab_hbm, out_hbm,
          table_v, src_v, lab_v, vt_v, ab_v, out_v, sem):
    wid = lax.axis_index("s")
    base = wid * _BPW
    c0 = pltpu.async_copy(table_hbm, table_v, sem)
    c1 = pltpu.async_copy(src_hbm.at[pl.ds(base, _BPW)], src_v, sem)
    c2 = pltpu.async_copy(lab_hbm.at[pl.ds(base, _BPW)], lab_v, sem)
    c3 = pltpu.async_copy(vt_hbm.at[pl.ds(base, _BPW)], vt_v, sem)
    c4 = pltpu.async_copy(ab_hbm.at[pl.ds(base, _BPW)], ab_v, sem)
    c0.wait()
    c1.wait()
    c2.wait()
    c3.wait()
    c4.wait()

    def step(i, carry):
        off = i * _LANES
        s = src_v[pl.ds(off, _LANES)]
        l = lab_v[pl.ds(off, _LANES)]
        v = vt_v[pl.ds(off, _LANES)]
        a = ab_v[pl.ds(off, _LANES)]
        idx = ((s * L + l) * V + v) * A + a
        out_v[pl.ds(off, _LANES)] = plsc.load_gather(table_v, [idx])
        return carry

    half = _BPW // 2
    lax.fori_loop(0, _STEPS // 2, step, 0, unroll=8)
    h0 = pltpu.async_copy(out_v.at[pl.ds(0, half)],
                          out_hbm.at[pl.ds(base, half)], sem)
    lax.fori_loop(_STEPS // 2, _STEPS, step, 0, unroll=8)
    h1 = pltpu.async_copy(out_v.at[pl.ds(half, half)],
                          out_hbm.at[pl.ds(base + half, half)], sem)
    h0.wait()
    h1.wait()


_balancer = functools.partial(
    pl.kernel,
    out_type=jax.ShapeDtypeStruct((B,), jnp.float32),
    mesh=plsc.VectorSubcoreMesh(core_axis_name="c", subcore_axis_name="s",
                                num_cores=1),
    compiler_params=pltpu.CompilerParams(needs_layout_passes=False),
    scratch_types=[
        pltpu.VMEM((TABLE,), jnp.float32),
        pltpu.VMEM((_BPW,), jnp.int32),
        pltpu.VMEM((_BPW,), jnp.int32),
        pltpu.VMEM((_BPW,), jnp.int32),
        pltpu.VMEM((_BPW,), jnp.int32),
        pltpu.VMEM((_BPW,), jnp.float32),
        pltpu.SemaphoreType.DMA,
    ],
)(_body)


@jax.jit
def kernel(label_balancing_weights_slva, sources, labels, variant_types,
           alt_count_bins):
    table = jnp.reshape(label_balancing_weights_slva, (TABLE,))
    return _balancer(
        table,
        sources.astype(jnp.int32),
        labels.astype(jnp.int32),
        variant_types.astype(jnp.int32),
        alt_count_bins.astype(jnp.int32),
    )


# speedup vs baseline: 1.0130x; 1.0130x over previous
import functools

import jax
import jax.numpy as jnp
from jax import lax
from jax.experimental import pallas as pl
from jax.experimental.pallas import tpu as pltpu
from jax.experimental.pallas import tpu_sc as plsc

S, L, V, A = 4, 3, 8, 10
B = 16384
TABLE = S * L * V * A

_info = plsc.get_sparse_core_info()
_NS, _LANES = _info.num_subcores, _info.num_lanes
_NW = _NS
_BPW = B // _NW
_STEPS = _BPW // _LANES


def _body(table_hbm, src_hbm, lab_hbm, vt_hbm, ab_hbm, out_hbm,
          table_v, src_v, lab_v, vt_v, ab_v, out_v, sem):
    wid = lax.axis_index("s")
    base = wid * _BPW
    c0 = pltpu.async_copy(table_hbm, table_v, sem)
    c1 = pltpu.async_copy(src_hbm.at[pl.ds(base, _BPW)], src_v, sem)
    c2 = pltpu.async_copy(lab_hbm.at[pl.ds(base, _BPW)], lab_v, sem)
    c3 = pltpu.async_copy(vt_hbm.at[pl.ds(base, _BPW)], vt_v, sem)
    c4 = pltpu.async_copy(ab_hbm.at[pl.ds(base, _BPW)], ab_v, sem)
    c0.wait()
    c1.wait()
    c2.wait()
    c3.wait()
    c4.wait()

    def step(i, carry):
        off = i * _LANES
        s = src_v[pl.ds(off, _LANES)]
        l = lab_v[pl.ds(off, _LANES)]
        v = vt_v[pl.ds(off, _LANES)]
        a = ab_v[pl.ds(off, _LANES)]
        idx = ((s * L + l) * V + v) * A + a
        out_v[pl.ds(off, _LANES)] = plsc.load_gather(table_v, [idx])
        return carry

    lax.fori_loop(0, _STEPS, step, 0, unroll=8)
    pltpu.sync_copy(out_v, out_hbm.at[pl.ds(base, _BPW)])


_balancer = functools.partial(
    pl.kernel,
    out_type=jax.ShapeDtypeStruct((B,), jnp.float32),
    mesh=plsc.VectorSubcoreMesh(core_axis_name="c", subcore_axis_name="s",
                                num_cores=1),
    compiler_params=pltpu.CompilerParams(needs_layout_passes=False),
    scratch_types=[
        pltpu.VMEM((TABLE,), jnp.float32),
        pltpu.VMEM((_BPW,), jnp.int32),
        pltpu.VMEM((_BPW,), jnp.int32),
        pltpu.VMEM((_BPW,), jnp.int32),
        pltpu.VMEM((_BPW,), jnp.int32),
        pltpu.VMEM((_BPW,), jnp.float32),
        pltpu.SemaphoreType.DMA,
    ],
)(_body)


@jax.jit
def kernel(label_balancing_weights_slva, sources, labels, variant_types,
           alt_count_bins):
    table = jnp.reshape(label_balancing_weights_slva, (TABLE,))
    return _balancer(
        table,
        sources.astype(jnp.int32),
        labels.astype(jnp.int32),
        variant_types.astype(jnp.int32),
        alt_count_bins.astype(jnp.int32),
    )
